# baseline (speedup 1.0000x reference)
import jax
import jax.numpy as jnp
from jax import lax
from jax.experimental import pallas as pl
from jax.experimental.pallas import tpu as pltpu

NN = 30
NP = 32
IN = 4
HID = 64
EH = 96
LAT = 16
NMP = 3
ALPHA = 0.2

BB = 8


_HI = lax.Precision.HIGHEST


def _lrelu(v):
    return jnp.where(v >= 0, v, ALPHA * v)


def _encoder_body(x_ref, w0a_ref, w0b_ref, w0d_ref, b0_ref, w1_ref, b1_ref,
                  nwh_ref, nwa_ref, nb0_ref, nw1_ref, nb1_ref,
                  wout_ref, bout_ref, out_ref):
    B = x_ref.shape[0]
    M = B * NP
    E = B * NP * NP
    h2 = x_ref[...].reshape(M, HID)
    selj = (lax.broadcasted_iota(jnp.int32, (NP * NP, NP), 0) // NP
            == lax.broadcasted_iota(jnp.int32, (NP * NP, NP), 1)
            ).astype(jnp.float32)
    for r in range(NMP):
        w0a = w0a_ref[r]
        w0b = w0b_ref[r]
        w0d = w0d_ref[r]
        b0 = b0_ref[r]
        w1 = w1_ref[r]
        b1 = b1_ref[r]
        nwh = nwh_ref[r]
        nwa = nwa_ref[r]
        nb0 = nb0_ref[r]
        nw1 = nw1_ref[r]
        nb1 = nb1_ref[r]

        n2 = jnp.sum(h2 * h2, axis=1, keepdims=True)
        P2 = jnp.dot(h2, w0a, precision=_HI) + n2 * w0d + b0
        Q2 = jnp.dot(h2, w0b, precision=_HI) + n2 * w0d
        h3 = h2.reshape(B, NP, HID)
        G3 = lax.dot_general(h3, h3, (((2,), (2,)), ((0,), (0,))),
                             precision=lax.Precision.HIGHEST)

        ppart = pltpu.repeat(P2.reshape(B, NP, EH), NP, axis=1)
        q3 = Q2.reshape(B, NP, EH)
        qpart = jnp.concatenate(
            [jnp.dot(selj, q3[b], precision=_HI) for b in range(B)], axis=0)
        r2 = pltpu.repeat(G3, NP, axis=1)
        gl = (r2 * selj[None]).reshape(E, NP)
        w2 = jnp.broadcast_to(-2.0 * w0d, (NP, EH))
        gpart = jnp.dot(gl, w2, precision=_HI)

        pre = ppart.reshape(E, EH) + qpart + gpart
        e2 = _lrelu(pre)
        f2 = _lrelu(jnp.dot(e2, w1, precision=_HI) + b1)
        f4 = f2.reshape(B, NP, NP, EH)
        agg = jnp.sum(f4[:, :NN], axis=1).reshape(M, EH)
        n1 = _lrelu(jnp.dot(h2, nwh, precision=_HI) + jnp.dot(agg, nwa, precision=_HI) + nb0)
        h2 = _lrelu(jnp.dot(n1, nw1, precision=_HI) + nb1)

    lat = jnp.dot(h2, wout_ref[...], precision=_HI) + bout_ref[...]
    out_ref[...] = lat.reshape(B, NP, LAT)[:, :NN, :]


def _mix_body(lat_ref, wm_ref, out_ref):
    out_ref[...] = jnp.dot(lat_ref[...], wm_ref[...], precision=_HI)


def kernel(x, params):
    bs = x.shape[0]
    h0 = jnp.pad(x, ((0, 0), (0, NP - NN), (0, HID - IN)))

    w0a = jnp.stack([params['ew%d_0' % i][:, :HID].T for i in range(NMP)])
    w0b = jnp.stack([params['ew%d_0' % i][:, HID:2 * HID].T for i in range(NMP)])
    w0d = jnp.stack([params['ew%d_0' % i][:, 2 * HID:].T for i in range(NMP)])
    b0 = jnp.stack([params['eb%d_0' % i][None] for i in range(NMP)])
    w1 = jnp.stack([params['ew%d_1' % i].T for i in range(NMP)])
    b1 = jnp.stack([params['eb%d_1' % i][None] for i in range(NMP)])
    nwh = jnp.stack([params['nw%d_0' % i][:, :HID].T for i in range(NMP)])
    nwa = jnp.stack([params['nw%d_0' % i][:, HID:].T for i in range(NMP)])
    nb0 = jnp.stack([params['nb%d_0' % i][None] for i in range(NMP)])
    nw1 = jnp.stack([params['nw%d_1' % i].T for i in range(NMP)])
    nb1 = jnp.stack([params['nb%d_1' % i][None] for i in range(NMP)])
    wout = params['w_out'].T
    bout = params['b_out'][None]
    wmix = params['w_mix'].T

    full = lambda s: pl.BlockSpec(s, lambda i: (0,) * len(s))
    lat = pl.pallas_call(
        _encoder_body,
        grid=(bs // BB,),
        in_specs=[
            pl.BlockSpec((BB, NP, HID), lambda i: (i, 0, 0)),
            full((NMP, HID, EH)), full((NMP, HID, EH)), full((NMP, 1, EH)),
            full((NMP, 1, EH)), full((NMP, EH, EH)), full((NMP, 1, EH)),
            full((NMP, HID, HID)), full((NMP, EH, HID)), full((NMP, 1, HID)),
            full((NMP, HID, HID)), full((NMP, 1, HID)),
            full((HID, LAT)), full((1, LAT)),
        ],
        out_specs=pl.BlockSpec((BB, NN, LAT), lambda i: (i, 0, 0)),
        out_shape=jax.ShapeDtypeStruct((bs, NN, LAT), jnp.float32),
        compiler_params=pltpu.CompilerParams(
            dimension_semantics=("arbitrary",),
        ),
    )(h0, w0a, w0b, w0d, b0, w1, b1, nwh, nwa, nb0, nw1, nb1, wout, bout)

    z = pl.pallas_call(
        _mix_body,
        out_shape=jax.ShapeDtypeStruct((bs, LAT), jnp.float32),
    )(lat.reshape(bs, NN * LAT), wmix)
    return z[None]


# speedup vs baseline: 6.4836x; 6.4836x over previous
import jax
import jax.numpy as jnp
from jax import lax
from jax.experimental import pallas as pl
from jax.experimental.pallas import tpu as pltpu

NN = 30
NP = 32
IN = 4
HID = 64
EH = 96
LAT = 16
NMP = 3
ALPHA = 0.2

BB = 8


def _lrelu(v):
    return jnp.where(v >= 0, v, ALPHA * v)


def _encoder_body(x_ref, w0a_ref, w0b_ref, w0d_ref, b0_ref, w1_ref, b1_ref,
                  nwh_ref, nwa_ref, nb0_ref, nw1_ref, nb1_ref,
                  wout_ref, bout_ref, out_ref):
    B = x_ref.shape[0]
    M = B * NP
    E = B * NP * NP
    h2 = x_ref[...].reshape(M, HID)
    selj = (lax.broadcasted_iota(jnp.int32, (NP * NP, NP), 0) // NP
            == lax.broadcasted_iota(jnp.int32, (NP * NP, NP), 1)
            ).astype(jnp.float32)
    for r in range(NMP):
        w0a = w0a_ref[r]
        w0b = w0b_ref[r]
        w0d = w0d_ref[r]
        b0 = b0_ref[r]
        w1 = w1_ref[r]
        b1 = b1_ref[r]
        nwh = nwh_ref[r]
        nwa = nwa_ref[r]
        nb0 = nb0_ref[r]
        nw1 = nw1_ref[r]
        nb1 = nb1_ref[r]

        n2 = jnp.sum(h2 * h2, axis=1, keepdims=True)
        P2 = jnp.dot(h2, w0a) + n2 * w0d + b0
        Q2 = jnp.dot(h2, w0b) + n2 * w0d
        h3 = h2.reshape(B, NP, HID)
        G3 = lax.dot_general(h3, h3, (((2,), (2,)), ((0,), (0,))),
                             precision=lax.Precision.HIGHEST)

        ppart = pltpu.repeat(P2.reshape(B, NP, EH), NP, axis=1)
        q3 = Q2.reshape(B, NP, EH)
        qpart = jnp.concatenate(
            [jnp.dot(selj, q3[b]) for b in range(B)], axis=0)
        r2 = pltpu.repeat(G3, NP, axis=1)
        gl = (r2 * selj[None]).reshape(E, NP)
        w2 = jnp.broadcast_to(-2.0 * w0d, (NP, EH))
        gpart = jnp.dot(gl, w2)

        pre = ppart.reshape(E, EH) + qpart + gpart
        e2 = _lrelu(pre)
        f2 = _lrelu(jnp.dot(e2, w1) + b1)
        f4 = f2.reshape(B, NP, NP, EH)
        agg = jnp.sum(f4[:, :NN], axis=1).reshape(M, EH)
        n1 = _lrelu(jnp.dot(h2, nwh) + jnp.dot(agg, nwa) + nb0)
        h2 = _lrelu(jnp.dot(n1, nw1) + nb1)

    lat = jnp.dot(h2, wout_ref[...]) + bout_ref[...]
    out_ref[...] = lat.reshape(B, NP, LAT)[:, :NN, :]


def _mix_body(lat_ref, wm_ref, out_ref):
    out_ref[...] = jnp.dot(lat_ref[...], wm_ref[...])


def kernel(x, params):
    bs = x.shape[0]
    h0 = jnp.pad(x, ((0, 0), (0, NP - NN), (0, HID - IN)))

    w0a = jnp.stack([params['ew%d_0' % i][:, :HID].T for i in range(NMP)])
    w0b = jnp.stack([params['ew%d_0' % i][:, HID:2 * HID].T for i in range(NMP)])
    w0d = jnp.stack([params['ew%d_0' % i][:, 2 * HID:].T for i in range(NMP)])
    b0 = jnp.stack([params['eb%d_0' % i][None] for i in range(NMP)])
    w1 = jnp.stack([params['ew%d_1' % i].T for i in range(NMP)])
    b1 = jnp.stack([params['eb%d_1' % i][None] for i in range(NMP)])
    nwh = jnp.stack([params['nw%d_0' % i][:, :HID].T for i in range(NMP)])
    nwa = jnp.stack([params['nw%d_0' % i][:, HID:].T for i in range(NMP)])
    nb0 = jnp.stack([params['nb%d_0' % i][None] for i in range(NMP)])
    nw1 = jnp.stack([params['nw%d_1' % i].T for i in range(NMP)])
    nb1 = jnp.stack([params['nb%d_1' % i][None] for i in range(NMP)])
    wout = params['w_out'].T
    bout = params['b_out'][None]
    wmix = params['w_mix'].T

    full = lambda s: pl.BlockSpec(s, lambda i: (0,) * len(s))
    lat = pl.pallas_call(
        _encoder_body,
        grid=(bs // BB,),
        in_specs=[
            pl.BlockSpec((BB, NP, HID), lambda i: (i, 0, 0)),
            full((NMP, HID, EH)), full((NMP, HID, EH)), full((NMP, 1, EH)),
            full((NMP, 1, EH)), full((NMP, EH, EH)), full((NMP, 1, EH)),
            full((NMP, HID, HID)), full((NMP, EH, HID)), full((NMP, 1, HID)),
            full((NMP, HID, HID)), full((NMP, 1, HID)),
            full((HID, LAT)), full((1, LAT)),
        ],
        out_specs=pl.BlockSpec((BB, NN, LAT), lambda i: (i, 0, 0)),
        out_shape=jax.ShapeDtypeStruct((bs, NN, LAT), jnp.float32),
        compiler_params=pltpu.CompilerParams(
            dimension_semantics=("arbitrary",),
        ),
    )(h0, w0a, w0b, w0d, b0, w1, b1, nwh, nwa, nb0, nw1, nb1, wout, bout)

    z = pl.pallas_call(
        _mix_body,
        out_shape=jax.ShapeDtypeStruct((bs, LAT), jnp.float32),
    )(lat.reshape(bs, NN * LAT), wmix)
    return z[None]


# speedup vs baseline: 7.3796x; 1.1382x over previous
import jax
import jax.numpy as jnp
from jax import lax
from jax.experimental import pallas as pl
from jax.experimental.pallas import tpu as pltpu

NN = 30
NP = 32
IN = 4
HID = 64
EH = 96
LAT = 16
NMP = 3
ALPHA = 0.2

BB = 16


def _lrelu(v):
    return jnp.maximum(v, ALPHA * v)


def _encoder_body(x_ref, w0a_ref, w0b_ref, w0d_ref, b0_ref, w1_ref, b1_ref,
                  nwh_ref, nwa_ref, nb0_ref, nw1_ref, nb1_ref,
                  wout_ref, bout_ref, out_ref):
    B = x_ref.shape[0]
    M = B * NP
    E = B * NP * NP
    h2 = x_ref[...].reshape(M, HID)
    selj = (lax.broadcasted_iota(jnp.int32, (NP * NP, NP), 0) // NP
            == lax.broadcasted_iota(jnp.int32, (NP * NP, NP), 1)
            ).astype(jnp.float32)
    for r in range(NMP):
        w0a = w0a_ref[r]
        w0b = w0b_ref[r]
        w0d = w0d_ref[r]
        b0 = b0_ref[r]
        w1 = w1_ref[r]
        b1 = b1_ref[r]
        nwh = nwh_ref[r]
        nwa = nwa_ref[r]
        nb0 = nb0_ref[r]
        nw1 = nw1_ref[r]
        nb1 = nb1_ref[r]

        n2 = jnp.sum(h2 * h2, axis=1, keepdims=True)
        P2 = jnp.dot(h2, w0a) + n2 * w0d + b0
        Q2 = jnp.dot(h2, w0b) + n2 * w0d
        h3 = h2.reshape(B, NP, HID)
        G3 = lax.dot_general(h3, h3, (((2,), (2,)), ((0,), (0,))),
                             precision=lax.Precision.HIGHEST)

        ppart = pltpu.repeat(P2.reshape(B, NP, EH), NP, axis=1)
        q3 = Q2.reshape(B, NP, EH)
        qpart = jnp.concatenate(
            [jnp.dot(selj, q3[b]) for b in range(B)], axis=0)
        r2 = pltpu.repeat(G3, NP, axis=1)
        gl = (r2 * selj[None]).reshape(E, NP)
        w2 = jnp.broadcast_to(-2.0 * w0d, (NP, EH))
        gpart = jnp.dot(gl, w2)

        pre = ppart.reshape(E, EH) + qpart + gpart
        e2 = _lrelu(pre)
        f2 = _lrelu(jnp.dot(e2, w1) + b1)
        f4 = f2.reshape(B, NP, NP, EH)
        agg = jnp.sum(f4[:, :NN], axis=1).reshape(M, EH)
        n1 = _lrelu(jnp.dot(h2, nwh) + jnp.dot(agg, nwa) + nb0)
        h2 = _lrelu(jnp.dot(n1, nw1) + nb1)

    lat = jnp.dot(h2, wout_ref[...]) + bout_ref[...]
    out_ref[...] = lat.reshape(B, NP, LAT)[:, :NN, :]


def _mix_body(lat_ref, wm_ref, out_ref):
    out_ref[...] = jnp.dot(lat_ref[...], wm_ref[...])


def kernel(x, params):
    bs = x.shape[0]
    h0 = jnp.pad(x, ((0, 0), (0, NP - NN), (0, HID - IN)))

    w0a = jnp.stack([params['ew%d_0' % i][:, :HID].T for i in range(NMP)])
    w0b = jnp.stack([params['ew%d_0' % i][:, HID:2 * HID].T for i in range(NMP)])
    w0d = jnp.stack([params['ew%d_0' % i][:, 2 * HID:].T for i in range(NMP)])
    b0 = jnp.stack([params['eb%d_0' % i][None] for i in range(NMP)])
    w1 = jnp.stack([params['ew%d_1' % i].T for i in range(NMP)])
    b1 = jnp.stack([params['eb%d_1' % i][None] for i in range(NMP)])
    nwh = jnp.stack([params['nw%d_0' % i][:, :HID].T for i in range(NMP)])
    nwa = jnp.stack([params['nw%d_0' % i][:, HID:].T for i in range(NMP)])
    nb0 = jnp.stack([params['nb%d_0' % i][None] for i in range(NMP)])
    nw1 = jnp.stack([params['nw%d_1' % i].T for i in range(NMP)])
    nb1 = jnp.stack([params['nb%d_1' % i][None] for i in range(NMP)])
    wout = params['w_out'].T
    bout = params['b_out'][None]
    wmix = params['w_mix'].T

    full = lambda s: pl.BlockSpec(s, lambda i: (0,) * len(s))
    lat = pl.pallas_call(
        _encoder_body,
        grid=(bs // BB,),
        in_specs=[
            pl.BlockSpec((BB, NP, HID), lambda i: (i, 0, 0)),
            full((NMP, HID, EH)), full((NMP, HID, EH)), full((NMP, 1, EH)),
            full((NMP, 1, EH)), full((NMP, EH, EH)), full((NMP, 1, EH)),
            full((NMP, HID, HID)), full((NMP, EH, HID)), full((NMP, 1, HID)),
            full((NMP, HID, HID)), full((NMP, 1, HID)),
            full((HID, LAT)), full((1, LAT)),
        ],
        out_specs=pl.BlockSpec((BB, NN, LAT), lambda i: (i, 0, 0)),
        out_shape=jax.ShapeDtypeStruct((bs, NN, LAT), jnp.float32),
        compiler_params=pltpu.CompilerParams(
            dimension_semantics=("parallel",),
        ),
    )(h0, w0a, w0b, w0d, b0, w1, b1, nwh, nwa, nb0, nw1, nb1, wout, bout)

    z = pl.pallas_call(
        _mix_body,
        out_shape=jax.ShapeDtypeStruct((bs, LAT), jnp.float32),
    )(lat.reshape(bs, NN * LAT), wmix)
    return z[None]
